# baseline (speedup 1.0000x reference)
import functools

import jax
import jax.numpy as jnp
from jax import lax
from jax.experimental import pallas as pl
from jax.experimental.pallas import tpu as pltpu
from jax.experimental.pallas import tpu_sc as plsc

H = 256
NC, NS = 2, 16
NW = NC * NS
C = 128


@functools.partial(jax.jit, static_argnums=(2, 3))
def _sc_lookup(ids, table_flat, BT, S):
    n_per_w = BT // NW
    nchunks = n_per_w // C
    rows_per_chunk = C // S
    mesh = plsc.VectorSubcoreMesh(
        core_axis_name="c", subcore_axis_name="s",
        num_cores=NC, num_subcores=NS,
    )

    @functools.partial(
        pl.kernel,
        out_type=jax.ShapeDtypeStruct((BT // S, S, 1, H), jnp.float32),
        mesh=mesh,
        compiler_params=pltpu.CompilerParams(needs_layout_passes=False),
        scratch_types=[
            pltpu.VMEM((H * 3,), jnp.float32),
            pltpu.VMEM((n_per_w,), jnp.int32),
            pltpu.VMEM((rows_per_chunk, S, 1, H), jnp.float32),
            pltpu.VMEM((rows_per_chunk, S, 1, H), jnp.float32),
            pltpu.SemaphoreType.DMA,
            pltpu.SemaphoreType.DMA,
        ],
    )
    def k(idx_hbm, table_hbm, out_hbm, tbl_v, ids_v, buf_a, buf_b, sem_a, sem_b):
        wid = lax.axis_index("s") * NC + lax.axis_index("c")
        base = wid * n_per_w
        lane = lax.iota(jnp.int32, 16)

        pltpu.sync_copy(table_hbm, tbl_v)
        pltpu.sync_copy(idx_hbm.at[pl.ds(base, n_per_w)], ids_v)

        def construct(g, buf):
            goff = g * C

            @plsc.parallel_loop(0, C, unroll=8)
            def _(t):
                row = plsc.load_gather(
                    ids_v, [jnp.full((16,), goff + t, jnp.int32)]) << 8
                for j in range(16):
                    buf[t // S, t % S, 0, pl.ds(j * 16, 16)] = plsc.load_gather(
                        tbl_v, [row + (j * 16) + lane])

        def start_out(g, buf, sem):
            r0 = (base + g * C) // S
            pltpu.async_copy(buf, out_hbm.at[pl.ds(r0, rows_per_chunk)], sem)

        def wait_out(buf, sem):
            pltpu.make_async_copy(
                buf, out_hbm.at[pl.ds(base // S, rows_per_chunk)], sem).wait()

        construct(0, buf_a)
        start_out(0, buf_a, sem_a)
        construct(1, buf_b)
        start_out(1, buf_b, sem_b)

        def body(p, carry):
            g1 = 2 * p + 2
            wait_out(buf_a, sem_a)
            construct(g1, buf_a)
            start_out(g1, buf_a, sem_a)
            wait_out(buf_b, sem_b)
            construct(g1 + 1, buf_b)
            start_out(g1 + 1, buf_b, sem_b)
            return carry

        lax.fori_loop(0, (nchunks - 2) // 2, body, 0)
        wait_out(buf_a, sem_a)
        wait_out(buf_b, sem_b)

    return k(ids, table_flat)


def kernel(input_ids, table):
    BT = input_ids.size
    S = input_ids.shape[1]
    ids = input_ids.reshape(BT).astype(jnp.int32)
    out = _sc_lookup(ids, table.reshape(-1), BT, S)
    return out.reshape(*input_ids.shape, table.shape[1])


# speedup vs baseline: 1.3173x; 1.3173x over previous
import functools

import jax
import jax.numpy as jnp
from jax import lax
from jax.experimental import pallas as pl
from jax.experimental.pallas import tpu as pltpu
from jax.experimental.pallas import tpu_sc as plsc

H = 256
NC, NS = 2, 16
NW = NC * NS
C = 128


@functools.partial(jax.jit, static_argnums=(2, 3))
def _sc_lookup(ids, table_flat, BT, S):
    n_per_w = BT // NW
    nchunks = n_per_w // C
    rows_per_chunk = C // S
    mesh = plsc.VectorSubcoreMesh(
        core_axis_name="c", subcore_axis_name="s",
        num_cores=NC, num_subcores=NS,
    )

    @functools.partial(
        pl.kernel,
        out_type=jax.ShapeDtypeStruct((BT // S, S, 1, H), jnp.float32),
        mesh=mesh,
        compiler_params=pltpu.CompilerParams(needs_layout_passes=False),
        scratch_types=[
            pltpu.VMEM((H * 3,), jnp.float32),
            pltpu.VMEM((n_per_w,), jnp.int32),
            pltpu.VMEM((rows_per_chunk, S, 1, H), jnp.float32),
            pltpu.VMEM((rows_per_chunk, S, 1, H), jnp.float32),
            pltpu.SemaphoreType.DMA,
            pltpu.SemaphoreType.DMA,
        ],
    )
    def k(idx_hbm, table_hbm, out_hbm, tbl_v, ids_v, buf_a, buf_b, sem_a, sem_b):
        wid = lax.axis_index("s") * NC + lax.axis_index("c")
        base = wid * n_per_w
        lane = lax.iota(jnp.int32, 16)

        pltpu.sync_copy(table_hbm, tbl_v)
        pltpu.sync_copy(idx_hbm.at[pl.ds(base, n_per_w)], ids_v)

        def construct(g, buf):
            goff = g * C

            @plsc.parallel_loop(0, C, unroll=2)
            def _(t):
                row = plsc.load_gather(
                    ids_v, [jnp.full((16,), goff + t, jnp.int32)]) << 8
                for j in range(16):
                    buf[t // S, t % S, 0, pl.ds(j * 16, 16)] = plsc.load_gather(
                        tbl_v, [row + (j * 16) + lane])

        def start_out(g, buf, sem):
            r0 = (base + g * C) // S
            pltpu.async_copy(buf, out_hbm.at[pl.ds(r0, rows_per_chunk)], sem)

        def wait_out(buf, sem):
            pltpu.make_async_copy(
                buf, out_hbm.at[pl.ds(base // S, rows_per_chunk)], sem).wait()

        construct(0, buf_a)
        start_out(0, buf_a, sem_a)
        construct(1, buf_b)
        start_out(1, buf_b, sem_b)

        def body(p, carry):
            g1 = 2 * p + 2
            wait_out(buf_a, sem_a)
            construct(g1, buf_a)
            start_out(g1, buf_a, sem_a)
            wait_out(buf_b, sem_b)
            construct(g1 + 1, buf_b)
            start_out(g1 + 1, buf_b, sem_b)
            return carry

        lax.fori_loop(0, (nchunks - 2) // 2, body, 0)
        wait_out(buf_a, sem_a)
        wait_out(buf_b, sem_b)

    return k(ids, table_flat)


def kernel(input_ids, table):
    BT = input_ids.size
    S = input_ids.shape[1]
    ids = input_ids.reshape(BT).astype(jnp.int32)
    out = _sc_lookup(ids, table.reshape(-1), BT, S)
    return out.reshape(*input_ids.shape, table.shape[1])
